# Initial kernel scaffold; baseline (speedup 1.0000x reference)
import jax
import jax.numpy as jnp
from jax.experimental import pallas as pl


def kernel(noisy, deno, curr_epoch):
    raise NotImplementedError("write your pallas kernel here")


# speedup vs baseline: 12.5731x; 12.5731x over previous
import jax
import jax.numpy as jnp
from jax.experimental import pallas as pl
from jax.experimental.pallas import tpu as pltpu

_T = 2
_NH = 64
_NW = 64
_L = 225
_NSEL = 9
_BIG = 1e30
_DENOM = 1.0 / (_T * _NH * _NW * _NSEL)


def _dnls_body(next_ref, dext_ref, out_ref, sn_ref, sd_ref):
    step = pl.program_id(0)

    @pl.when(step < 225)
    def _build():
        a = step // 15
        b = step - a * 15

        ci = jax.lax.broadcasted_iota(jnp.int32, (384, 64), 0)
        qi = jax.lax.broadcasted_iota(jnp.int32, (384, 64), 1)
        rel = ci - 4 * qi
        msel = ((rel >= 0) & (rel <= 6)).astype(jnp.float32)

        for t in range(_T):
            dn = jnp.zeros((264, 384), jnp.float32)
            dd = jnp.zeros((264, 384), jnp.float32)
            for c in range(3):
                bn = next_ref[t, c, 7:271, 7:391]
                bd = dext_ref[t, c, 7:271, 7:391]
                full = next_ref[t, c]
                full = pltpu.roll(full, 280 - a, 0)
                full = pltpu.roll(full, 512 - b, 1)
                sh = full[0:264, 0:384]
                dn = dn + (bn - sh) * (bn - sh)
                dd = dd + (bd - sh) * (bd - sh)
            for buf, ref in ((dn, sn_ref), (dd, sd_ref)):
                v = buf[0:256].reshape(64, 4, 384)[:, 0, :]
                for i in range(1, 7):
                    v = v + buf[i:i + 256].reshape(64, 4, 384)[:, 0, :]
                slab = jnp.dot(v, msel,
                               preferred_element_type=jnp.float32,
                               precision=jax.lax.Precision.HIGHEST)
                ref[pl.ds(step, 1), t] = slab[None]

    @pl.when(step == 225)
    def _fixup():
        def body(i, _):
            ah = i // 15
            aw = i - ah * 15
            fh = (jnp.maximum(ah, 7), jnp.maximum(ah, 3), jnp.minimum(ah, 10))
            fw = (jnp.maximum(aw, 7), jnp.maximum(aw, 3), jnp.minimum(aw, 10))
            rc = (0, 1, 63)
            for ref in (sn_ref, sd_ref):
                for r, f in zip(rc, fh):
                    src = f * 15 + aw
                    ref[pl.ds(i, 1), :, r:r + 1, :] = \
                        ref[pl.ds(src, 1), :, r:r + 1, :]
                for cc, g in zip(rc, fw):
                    src = ah * 15 + g
                    ref[pl.ds(i, 1), :, :, cc:cc + 1] = \
                        ref[pl.ds(src, 1), :, :, cc:cc + 1]
                for r, f in zip(rc, fh):
                    for cc, g in zip(rc, fw):
                        src = f * 15 + g
                        ref[pl.ds(i, 1), :, r:r + 1, cc:cc + 1] = \
                            ref[pl.ds(src, 1), :, r:r + 1, cc:cc + 1]
            return 0
        jax.lax.fori_loop(0, 225, body, 0)

    @pl.when(step == 226)
    def _select():
        pi = jax.lax.broadcasted_iota(jnp.int32, (_L, 1, 1, 64), 0)
        prio = jnp.where(pi == 112, 0, jnp.where(pi < 112, pi + 1, pi))
        total = jnp.zeros((1, 1), jnp.float32)
        qblk = 8
        for cb in range(_NH // qblk):
            r0 = cb * qblk
            dv = sn_ref[:, :, r0:r0 + qblk, :]
            rv = sd_ref[:, :, r0:r0 + qblk, :]
            acc = jnp.zeros((_T, qblk, _NW), jnp.float32)
            for j in range(10):
                m = jnp.min(dv, axis=0, keepdims=True)
                eq = dv == m
                pq = jnp.where(eq, prio, 1000000)
                pm = jnp.min(pq, axis=0, keepdims=True)
                sel = pq == pm
                if j > 0:
                    acc = acc + jnp.sum(jnp.where(sel, rv, 0.0), axis=0)
                if j < 9:
                    dv = jnp.where(sel, _BIG, dv)
            total = total + jnp.sum(acc)[None, None]
        out_ref[...] = total * _DENOM


def kernel(noisy, deno, curr_epoch):
    del curr_epoch
    n = noisy[0].astype(jnp.float32)
    d = deno[0].astype(jnp.float32)
    pad = ((0, 0), (0, 0), (10, 10), (10, 10))
    npad = jnp.pad(n, pad, mode='reflect')
    dpad = jnp.pad(d, pad, mode='reflect')
    next_ext = jnp.zeros((_T, 3, 280, 512), jnp.float32)
    next_ext = next_ext.at[:, :, :276, :276].set(npad)
    dext = jnp.zeros((_T, 3, 280, 512), jnp.float32)
    dext = dext.at[:, :, :276, :276].set(dpad)

    res = pl.pallas_call(
        _dnls_body,
        grid=(227,),
        in_specs=[
            pl.BlockSpec((_T, 3, 280, 512), lambda i: (0, 0, 0, 0)),
            pl.BlockSpec((_T, 3, 280, 512), lambda i: (0, 0, 0, 0)),
        ],
        out_specs=pl.BlockSpec((1, 1), lambda i: (0, 0)),
        out_shape=jax.ShapeDtypeStruct((1, 1), jnp.float32),
        scratch_shapes=[
            pltpu.VMEM((_L, _T, _NH, _NW), jnp.float32),
            pltpu.VMEM((_L, _T, _NH, _NW), jnp.float32),
        ],
        compiler_params=pltpu.CompilerParams(
            dimension_semantics=("arbitrary",),
        ),
    )(next_ext, dext)
    return res[0, 0]


# speedup vs baseline: 17.6973x; 1.4075x over previous
import jax
import jax.numpy as jnp
from jax.experimental import pallas as pl
from jax.experimental.pallas import tpu as pltpu

_T = 2
_NH = 64
_NW = 64
_L = 225
_NSEL = 9
_BIG = 1e30
_DENOM = 1.0 / (_T * _NH * _NW * _NSEL)


def _dnls_body(next_ref, dext_ref, out_ref, rr_ref, sn_ref, sd_ref):
    step = pl.program_id(0)

    @pl.when(step < 15)
    def _build():
        a = step

        for t in range(_T):
            for c in range(3):
                rr_ref[t, c] = pltpu.roll(next_ref[t, c], 280 - a, 0)

        ci = jax.lax.broadcasted_iota(jnp.int32, (384, 64), 0)
        qi = jax.lax.broadcasted_iota(jnp.int32, (384, 64), 1)
        rel = ci - 4 * qi
        msel = ((rel >= 0) & (rel <= 6)).astype(jnp.float32)

        for b in range(15):
            for t in range(_T):
                dn = jnp.zeros((264, 384), jnp.float32)
                dd = jnp.zeros((264, 384), jnp.float32)
                for c in range(3):
                    bn = next_ref[t, c, 7:271, 7:391]
                    bd = dext_ref[t, c, 7:271, 7:391]
                    sh = rr_ref[t, c, 0:264, b:b + 384]
                    dn = dn + (bn - sh) * (bn - sh)
                    dd = dd + (bd - sh) * (bd - sh)
                for buf, ref in ((dn, sn_ref), (dd, sd_ref)):
                    v = buf[0:256].reshape(64, 4, 384)[:, 0, :]
                    for i in range(1, 7):
                        v = v + buf[i:i + 256].reshape(64, 4, 384)[:, 0, :]
                    slab = jnp.dot(v, msel,
                                   preferred_element_type=jnp.float32,
                                   precision=jax.lax.Precision.HIGHEST)
                    ref[pl.ds(15 * a + b, 1), :, t * 64:(t + 1) * 64] = \
                        slab[None]

    @pl.when(step == 15)
    def _fixup():
        def body(i, _):
            ah = i // 15
            aw = i - ah * 15
            fh = (jnp.maximum(ah, 7), jnp.maximum(ah, 3), jnp.minimum(ah, 10))
            fw = (jnp.maximum(aw, 7), jnp.maximum(aw, 3), jnp.minimum(aw, 10))
            rc = (0, 1, 63)
            for ref in (sn_ref, sd_ref):
                for r, f in zip(rc, fh):
                    src = f * 15 + aw
                    ref[pl.ds(i, 1), r:r + 1, :] = \
                        ref[pl.ds(src, 1), r:r + 1, :]
                for cc, g in zip(rc, fw):
                    src = ah * 15 + g
                    for t in range(_T):
                        q = t * 64 + cc
                        ref[pl.ds(i, 1), :, q:q + 1] = \
                            ref[pl.ds(src, 1), :, q:q + 1]
                for r, f in zip(rc, fh):
                    for cc, g in zip(rc, fw):
                        src = f * 15 + g
                        for t in range(_T):
                            q = t * 64 + cc
                            ref[pl.ds(i, 1), r:r + 1, q:q + 1] = \
                                ref[pl.ds(src, 1), r:r + 1, q:q + 1]
            return 0
        jax.lax.fori_loop(0, 225, body, 0)

    @pl.when(step == 16)
    def _select():
        total = jnp.zeros((1, 1), jnp.float32)
        qblk = 8
        for cb in range(_NH // qblk):
            r0 = cb * qblk

            def body(j, carry):
                ds_, rs_ = carry
                l = jnp.where(j == 0, 112, jnp.where(j <= 112, j - 1, j))
                dn = sn_ref[pl.ds(l, 1), r0:r0 + qblk, :][0]
                rn = sd_ref[pl.ds(l, 1), r0:r0 + qblk, :][0]
                new_d = []
                new_r = []
                for s in range(10):
                    less = dn < ds_[s]
                    new_d.append(jnp.where(less, dn, ds_[s]))
                    new_r.append(jnp.where(less, rn, rs_[s]))
                    dn = jnp.where(less, ds_[s], dn)
                    rn = jnp.where(less, rs_[s], rn)
                return tuple(new_d), tuple(new_r)

            init_d = tuple(jnp.full((qblk, 128), _BIG, jnp.float32)
                           for _ in range(10))
            init_r = tuple(jnp.zeros((qblk, 128), jnp.float32)
                           for _ in range(10))
            _, rs_ = jax.lax.fori_loop(0, _L, body, (init_d, init_r))
            acc = rs_[1]
            for s in range(2, 10):
                acc = acc + rs_[s]
            total = total + jnp.sum(acc)[None, None]
        out_ref[...] = total * _DENOM


def kernel(noisy, deno, curr_epoch):
    del curr_epoch
    n = noisy[0].astype(jnp.float32)
    d = deno[0].astype(jnp.float32)
    pad = ((0, 0), (0, 0), (10, 10), (10, 10))
    npad = jnp.pad(n, pad, mode='reflect')
    dpad = jnp.pad(d, pad, mode='reflect')
    next_ext = jnp.zeros((_T, 3, 280, 512), jnp.float32)
    next_ext = next_ext.at[:, :, :276, :276].set(npad)
    dext = jnp.zeros((_T, 3, 280, 512), jnp.float32)
    dext = dext.at[:, :, :276, :276].set(dpad)

    res = pl.pallas_call(
        _dnls_body,
        grid=(17,),
        in_specs=[
            pl.BlockSpec((_T, 3, 280, 512), lambda i: (0, 0, 0, 0)),
            pl.BlockSpec((_T, 3, 280, 512), lambda i: (0, 0, 0, 0)),
        ],
        out_specs=pl.BlockSpec((1, 1), lambda i: (0, 0)),
        out_shape=jax.ShapeDtypeStruct((1, 1), jnp.float32),
        scratch_shapes=[
            pltpu.VMEM((_T, 3, 280, 512), jnp.float32),
            pltpu.VMEM((_L, _NH, _T * _NW), jnp.float32),
            pltpu.VMEM((_L, _NH, _T * _NW), jnp.float32),
        ],
        compiler_params=pltpu.CompilerParams(
            dimension_semantics=("arbitrary",),
        ),
    )(next_ext, dext)
    return res[0, 0]
